# Initial kernel scaffold; baseline (speedup 1.0000x reference)
import jax
import jax.numpy as jnp
from jax.experimental import pallas as pl


def kernel(x, edge_index, W_self1, b_self1, W_neigh1, W_self2, b_self2, W_neigh2):
    raise NotImplementedError("write your pallas kernel here")


# speedup vs baseline: 3.2340x; 3.2340x over previous
import functools

import jax
import jax.numpy as jnp
from jax import lax
from jax.experimental import pallas as pl
from jax.experimental.pallas import tpu as pltpu
from jax.experimental.pallas import tpu_sc as plsc

_N = 10000
_E = 320000
_D = 128
_CH = 128
_NC = 2
_NS = 16
_NT = _NC * _NS
_RPT = 80
_EPAD = _NT * _RPT * _CH
_NPAD = 10112
_RZ = _NPAD // _NS
_BD = 2048


def _sc_scatter(table, srcp, dstp, zeros):
    mesh = plsc.VectorSubcoreMesh(core_axis_name="c", subcore_axis_name="s")

    @functools.partial(
        pl.kernel,
        mesh=mesh,
        out_type=jax.ShapeDtypeStruct((_NC, _NPAD, _D), jnp.float32),
        scratch_types=[
            pltpu.VMEM((_RPT, _CH), jnp.int32),
            pltpu.VMEM((_RPT, _CH), jnp.int32),
            pltpu.VMEM((_CH, _D), jnp.float32),
            pltpu.VMEM_SHARED((_NPAD, _D), jnp.float32),
            pltpu.SemaphoreType.DMA,
        ],
    )
    def k(table_hbm, src_hbm, dst_hbm, zero_hbm, out_hbm,
          src_v, dst_v, rows_v, agg_sh, sem):
        c = lax.axis_index("c")
        s = lax.axis_index("s")
        wid = c * _NS + s
        pltpu.sync_copy(zero_hbm.at[pl.ds(s * _RZ, _RZ)],
                        agg_sh.at[pl.ds(s * _RZ, _RZ)])
        pltpu.sync_copy(src_hbm.at[pl.ds(wid * _RPT, _RPT)], src_v)
        pltpu.sync_copy(dst_hbm.at[pl.ds(wid * _RPT, _RPT)], dst_v)
        plsc.subcore_barrier()

        def body(j, carry):
            pltpu.async_copy(table_hbm.at[src_v.at[j]], rows_v, sem).wait()
            pltpu.sync_copy(rows_v, agg_sh.at[dst_v.at[j]], add=True)
            return carry

        lax.fori_loop(0, _RPT, body, 0)
        plsc.subcore_barrier()
        pltpu.sync_copy(agg_sh.at[pl.ds(s * _RZ, _RZ)],
                        out_hbm.at[c, pl.ds(s * _RZ, _RZ)])

    return k(table, srcp, dstp, zeros)


def _tc_degree(dst_col):
    steps = _EPAD // _BD

    def body(d_ref, out_ref):
        i = pl.program_id(0)
        d = d_ref[...]
        q = d // _D
        r = d - q * _D
        lanes = lax.broadcasted_iota(jnp.int32, (_BD, _D), 1)
        ohq = (q == lanes).astype(jnp.float32)
        ohr = (r == lanes).astype(jnp.float32)
        blk = lax.dot_general(ohq, ohr, (((0,), (0,)), ((), ())),
                              preferred_element_type=jnp.float32)

        @pl.when(i == 0)
        def _():
            out_ref[...] = jnp.zeros_like(out_ref)

        out_ref[...] += blk

        @pl.when(i == steps - 1)
        def _():
            out_ref[...] = jnp.maximum(out_ref[...], 1.0)

    return pl.pallas_call(
        body,
        grid=(steps,),
        in_specs=[pl.BlockSpec((_BD, 1), lambda i: (i, 0))],
        out_specs=pl.BlockSpec((_D, _D), lambda i: (0, 0)),
        out_shape=jax.ShapeDtypeStruct((_D, _D), jnp.float32),
    )(dst_col)


def _tc_first(x, WsT, b, WnT):
    bm = 2000
    grid = _N // bm

    def body(x_ref, ws_ref, b_ref, wn_ref, self_ref, tab_ref):
        h = x_ref[...]
        self_ref[...] = (
            jnp.dot(h, ws_ref[...], preferred_element_type=jnp.float32)
            + b_ref[...]
        )
        tab_ref[...] = jnp.dot(h, wn_ref[...], preferred_element_type=jnp.float32)

    return pl.pallas_call(
        body,
        grid=(grid,),
        in_specs=[
            pl.BlockSpec((bm, _D), lambda i: (i, 0)),
            pl.BlockSpec((_D, _D), lambda i: (0, 0)),
            pl.BlockSpec((1, _D), lambda i: (0, 0)),
            pl.BlockSpec((_D, _D), lambda i: (0, 0)),
        ],
        out_specs=[
            pl.BlockSpec((bm, _D), lambda i: (i, 0)),
            pl.BlockSpec((bm, _D), lambda i: (i, 0)),
        ],
        out_shape=[
            jax.ShapeDtypeStruct((_N, _D), jnp.float32),
            jax.ShapeDtypeStruct((_N, _D), jnp.float32),
        ],
    )(x, WsT, b, WnT)


def _tc_mid(self1, a0, a1, deg, WsT, b, WnT):
    bm = 2000
    grid = _N // bm

    def body(s_ref, a0_ref, a1_ref, deg_ref, ws_ref, b_ref, wn_ref,
             self_ref, tab_ref):
        agg = a0_ref[...] + a1_ref[...]
        h1 = jnp.maximum(s_ref[...] + agg / deg_ref[...], 0.0)
        self_ref[...] = (
            jnp.dot(h1, ws_ref[...], preferred_element_type=jnp.float32)
            + b_ref[...]
        )
        tab_ref[...] = jnp.dot(h1, wn_ref[...], preferred_element_type=jnp.float32)

    return pl.pallas_call(
        body,
        grid=(grid,),
        in_specs=[
            pl.BlockSpec((bm, _D), lambda i: (i, 0)),
            pl.BlockSpec((bm, _D), lambda i: (i, 0)),
            pl.BlockSpec((bm, _D), lambda i: (i, 0)),
            pl.BlockSpec((bm, 1), lambda i: (i, 0)),
            pl.BlockSpec((_D, _D), lambda i: (0, 0)),
            pl.BlockSpec((1, _D), lambda i: (0, 0)),
            pl.BlockSpec((_D, _D), lambda i: (0, 0)),
        ],
        out_specs=[
            pl.BlockSpec((bm, _D), lambda i: (i, 0)),
            pl.BlockSpec((bm, _D), lambda i: (i, 0)),
        ],
        out_shape=[
            jax.ShapeDtypeStruct((_N, _D), jnp.float32),
            jax.ShapeDtypeStruct((_N, _D), jnp.float32),
        ],
    )(self1, a0, a1, deg, WsT, b, WnT)


def _tc_final(self2, a0, a1, deg):
    bm = 2000
    grid = _N // bm

    def body(s_ref, a0_ref, a1_ref, deg_ref, out_ref):
        agg = a0_ref[...] + a1_ref[...]
        out_ref[...] = s_ref[...] + agg / deg_ref[...]

    return pl.pallas_call(
        body,
        grid=(grid,),
        in_specs=[
            pl.BlockSpec((bm, _D), lambda i: (i, 0)),
            pl.BlockSpec((bm, _D), lambda i: (i, 0)),
            pl.BlockSpec((bm, _D), lambda i: (i, 0)),
            pl.BlockSpec((bm, 1), lambda i: (i, 0)),
        ],
        out_specs=pl.BlockSpec((bm, _D), lambda i: (i, 0)),
        out_shape=jax.ShapeDtypeStruct((_N, _D), jnp.float32),
    )(self2, a0, a1, deg)


def kernel(x, edge_index, W_self1, b_self1, W_neigh1, W_self2, b_self2, W_neigh2):
    src = edge_index[0]
    dst = edge_index[1]
    pad = _EPAD - _E
    srcp = jnp.concatenate([src, jnp.zeros((pad,), jnp.int32)]).reshape(
        _NT * _RPT, _CH)
    dstp = jnp.concatenate([dst, jnp.full((pad,), _N, jnp.int32)]).reshape(
        _NT * _RPT, _CH)
    zeros = jnp.zeros((_NPAD, _D), jnp.float32)

    deg2d = _tc_degree(dstp.reshape(_EPAD, 1))
    deg = deg2d.reshape(_D * _D)[:_N].reshape(_N, 1)

    self1, tab1 = _tc_first(x, W_self1.T, b_self1.reshape(1, _D), W_neigh1.T)
    aggs1 = _sc_scatter(tab1, srcp, dstp, zeros)
    self2, tab2 = _tc_mid(self1, aggs1[0, :_N], aggs1[1, :_N], deg,
                          W_self2.T, b_self2.reshape(1, _D), W_neigh2.T)
    aggs2 = _sc_scatter(tab2, srcp, dstp, zeros)
    return _tc_final(self2, aggs2[0, :_N], aggs2[1, :_N], deg)


# speedup vs baseline: 3.5995x; 1.1130x over previous
import functools

import jax
import jax.numpy as jnp
from jax import lax
from jax.experimental import pallas as pl
from jax.experimental.pallas import tpu as pltpu
from jax.experimental.pallas import tpu_sc as plsc

_N = 10000
_E = 320000
_D = 128
_CH = 128
_NC = 2
_NS = 16
_NT = _NC * _NS
_RPT = 80
_EPAD = _NT * _RPT * _CH
_NPAD = 10112
_RZ = _NPAD // _NS
_BD = 2048


def _sc_scatter(table, srcp, dstp, zeros):
    mesh = plsc.VectorSubcoreMesh(core_axis_name="c", subcore_axis_name="s")

    nbuf = 2
    half = _RPT // 2

    @functools.partial(
        pl.kernel,
        mesh=mesh,
        out_type=jax.ShapeDtypeStruct((_NC, _NPAD, _D), jnp.float32),
        scratch_types=[
            pltpu.VMEM((half, _CH), jnp.int32),
            pltpu.VMEM((half, _CH), jnp.int32),
            pltpu.VMEM((nbuf, _CH, _D), jnp.float32),
            pltpu.VMEM_SHARED((_NPAD, _D), jnp.float32),
            pltpu.SemaphoreType.DMA,
        ],
    )
    def k(table_hbm, src_hbm, dst_hbm, zero_hbm, out_hbm,
          src_v, dst_v, rows_v, agg_sh, sem):
        c = lax.axis_index("c")
        s = lax.axis_index("s")
        wid = c * _NS + s
        pltpu.sync_copy(zero_hbm.at[pl.ds(s * _RZ, _RZ)],
                        agg_sh.at[pl.ds(s * _RZ, _RZ)])
        plsc.subcore_barrier()

        def gfire(j, b):
            pltpu.async_copy(table_hbm.at[src_v.at[j]], rows_v.at[b], sem)

        for h in range(2):
            base = wid * _RPT + h * half
            pltpu.sync_copy(src_hbm.at[pl.ds(base, half)], src_v)
            pltpu.sync_copy(dst_hbm.at[pl.ds(base, half)], dst_v)
            gfire(0, 0)

            def outer(g, carry):
                for b in range(nbuf):
                    j = g * nbuf + b

                    @pl.when(j + 1 < half)
                    def _():
                        gfire(j + 1, (b + 1) % nbuf)

                    pltpu.make_async_copy(
                        table_hbm.at[src_v.at[j]], rows_v.at[b], sem).wait()
                    pltpu.sync_copy(rows_v.at[b], agg_sh.at[dst_v.at[j]],
                                    add=True)
                return carry

            lax.fori_loop(0, half // nbuf, outer, 0)
        plsc.subcore_barrier()
        pltpu.sync_copy(agg_sh.at[pl.ds(s * _RZ, _RZ)],
                        out_hbm.at[c, pl.ds(s * _RZ, _RZ)])

    return k(table, srcp, dstp, zeros)


def _tc_degree(dst_col):
    steps = _EPAD // _BD

    def body(d_ref, out_ref):
        i = pl.program_id(0)
        d = d_ref[...]
        q = d // _D
        r = d - q * _D
        lanes = lax.broadcasted_iota(jnp.int32, (_BD, _D), 1)
        ohq = (q == lanes).astype(jnp.float32)
        ohr = (r == lanes).astype(jnp.float32)
        blk = lax.dot_general(ohq, ohr, (((0,), (0,)), ((), ())),
                              preferred_element_type=jnp.float32)

        @pl.when(i == 0)
        def _():
            out_ref[...] = jnp.zeros_like(out_ref)

        out_ref[...] += blk

        @pl.when(i == steps - 1)
        def _():
            out_ref[...] = jnp.maximum(out_ref[...], 1.0)

    return pl.pallas_call(
        body,
        grid=(steps,),
        in_specs=[pl.BlockSpec((_BD, 1), lambda i: (i, 0))],
        out_specs=pl.BlockSpec((_D, _D), lambda i: (0, 0)),
        out_shape=jax.ShapeDtypeStruct((_D, _D), jnp.float32),
    )(dst_col)


def _tc_first(x, WsT, b, WnT):
    bm = 2000
    grid = _N // bm

    def body(x_ref, ws_ref, b_ref, wn_ref, self_ref, tab_ref):
        h = x_ref[...]
        self_ref[...] = (
            jnp.dot(h, ws_ref[...], preferred_element_type=jnp.float32)
            + b_ref[...]
        )
        tab_ref[...] = jnp.dot(h, wn_ref[...], preferred_element_type=jnp.float32)

    return pl.pallas_call(
        body,
        grid=(grid,),
        in_specs=[
            pl.BlockSpec((bm, _D), lambda i: (i, 0)),
            pl.BlockSpec((_D, _D), lambda i: (0, 0)),
            pl.BlockSpec((1, _D), lambda i: (0, 0)),
            pl.BlockSpec((_D, _D), lambda i: (0, 0)),
        ],
        out_specs=[
            pl.BlockSpec((bm, _D), lambda i: (i, 0)),
            pl.BlockSpec((bm, _D), lambda i: (i, 0)),
        ],
        out_shape=[
            jax.ShapeDtypeStruct((_N, _D), jnp.float32),
            jax.ShapeDtypeStruct((_N, _D), jnp.float32),
        ],
    )(x, WsT, b, WnT)


def _tc_mid(self1, a0, a1, deg, WsT, b, WnT):
    bm = 2000
    grid = _N // bm

    def body(s_ref, a0_ref, a1_ref, deg_ref, ws_ref, b_ref, wn_ref,
             self_ref, tab_ref):
        agg = a0_ref[...] + a1_ref[...]
        h1 = jnp.maximum(s_ref[...] + agg / deg_ref[...], 0.0)
        self_ref[...] = (
            jnp.dot(h1, ws_ref[...], preferred_element_type=jnp.float32)
            + b_ref[...]
        )
        tab_ref[...] = jnp.dot(h1, wn_ref[...], preferred_element_type=jnp.float32)

    return pl.pallas_call(
        body,
        grid=(grid,),
        in_specs=[
            pl.BlockSpec((bm, _D), lambda i: (i, 0)),
            pl.BlockSpec((bm, _D), lambda i: (i, 0)),
            pl.BlockSpec((bm, _D), lambda i: (i, 0)),
            pl.BlockSpec((bm, 1), lambda i: (i, 0)),
            pl.BlockSpec((_D, _D), lambda i: (0, 0)),
            pl.BlockSpec((1, _D), lambda i: (0, 0)),
            pl.BlockSpec((_D, _D), lambda i: (0, 0)),
        ],
        out_specs=[
            pl.BlockSpec((bm, _D), lambda i: (i, 0)),
            pl.BlockSpec((bm, _D), lambda i: (i, 0)),
        ],
        out_shape=[
            jax.ShapeDtypeStruct((_N, _D), jnp.float32),
            jax.ShapeDtypeStruct((_N, _D), jnp.float32),
        ],
    )(self1, a0, a1, deg, WsT, b, WnT)


def _tc_final(self2, a0, a1, deg):
    bm = 2000
    grid = _N // bm

    def body(s_ref, a0_ref, a1_ref, deg_ref, out_ref):
        agg = a0_ref[...] + a1_ref[...]
        out_ref[...] = s_ref[...] + agg / deg_ref[...]

    return pl.pallas_call(
        body,
        grid=(grid,),
        in_specs=[
            pl.BlockSpec((bm, _D), lambda i: (i, 0)),
            pl.BlockSpec((bm, _D), lambda i: (i, 0)),
            pl.BlockSpec((bm, _D), lambda i: (i, 0)),
            pl.BlockSpec((bm, 1), lambda i: (i, 0)),
        ],
        out_specs=pl.BlockSpec((bm, _D), lambda i: (i, 0)),
        out_shape=jax.ShapeDtypeStruct((_N, _D), jnp.float32),
    )(self2, a0, a1, deg)


def kernel(x, edge_index, W_self1, b_self1, W_neigh1, W_self2, b_self2, W_neigh2):
    src = edge_index[0]
    dst = edge_index[1]
    pad = _EPAD - _E
    srcp = jnp.concatenate([src, jnp.zeros((pad,), jnp.int32)]).reshape(
        _NT * _RPT, _CH)
    dstp = jnp.concatenate([dst, jnp.full((pad,), _N, jnp.int32)]).reshape(
        _NT * _RPT, _CH)
    zeros = jnp.zeros((_NPAD, _D), jnp.float32)

    deg2d = _tc_degree(dstp.reshape(_EPAD, 1))
    deg = deg2d.reshape(_D * _D)[:_N].reshape(_N, 1)

    self1, tab1 = _tc_first(x, W_self1.T, b_self1.reshape(1, _D), W_neigh1.T)
    aggs1 = _sc_scatter(tab1, srcp, dstp, zeros)
    self2, tab2 = _tc_mid(self1, aggs1[0, :_N], aggs1[1, :_N], deg,
                          W_self2.T, b_self2.reshape(1, _D), W_neigh2.T)
    aggs2 = _sc_scatter(tab2, srcp, dstp, zeros)
    return _tc_final(self2, aggs2[0, :_N], aggs2[1, :_N], deg)


# speedup vs baseline: 3.9873x; 1.1077x over previous
import functools

import jax
import jax.numpy as jnp
from jax import lax
from jax.experimental import pallas as pl
from jax.experimental.pallas import tpu as pltpu
from jax.experimental.pallas import tpu_sc as plsc

_N = 10000
_E = 320000
_D = 128
_CH = 128
_NC = 2
_NS = 16
_NT = _NC * _NS
_RPT = 80
_EPAD = _NT * _RPT * _CH
_NPAD = 10112
_RZ = _NPAD // _NS
_BD = 2048


def _sc_scatter(table, srcp, dstp, zeros):
    mesh = plsc.VectorSubcoreMesh(core_axis_name="c", subcore_axis_name="s")

    nbuf = 2
    half = _RPT // 2

    @functools.partial(
        pl.kernel,
        mesh=mesh,
        out_type=jax.ShapeDtypeStruct((_NC, _NPAD, _D), jnp.float32),
        scratch_types=[
            pltpu.VMEM((half, _CH), jnp.int32),
            pltpu.VMEM((half, _CH), jnp.int32),
            pltpu.VMEM((nbuf, _CH, _D), jnp.float32),
            pltpu.VMEM_SHARED((_NPAD, _D), jnp.float32),
            pltpu.SemaphoreType.DMA,
        ],
    )
    def k(table_hbm, src_hbm, dst_hbm, zero_hbm, out_hbm,
          src_v, dst_v, rows_v, agg_sh, sem):
        c = lax.axis_index("c")
        s = lax.axis_index("s")
        wid = c * _NS + s
        pltpu.sync_copy(zero_hbm.at[pl.ds(s * _RZ, _RZ)],
                        agg_sh.at[pl.ds(s * _RZ, _RZ)])
        plsc.subcore_barrier()

        def gfire(j, b):
            pltpu.async_copy(table_hbm.at[src_v.at[j]], rows_v.at[b], sem)

        for h in range(2):
            base = wid * _RPT + h * half
            pltpu.sync_copy(src_hbm.at[pl.ds(base, half)], src_v)
            pltpu.sync_copy(dst_hbm.at[pl.ds(base, half)], dst_v)
            gfire(0, 0)

            def outer(g, carry):
                for b in range(nbuf):
                    j = g * nbuf + b

                    @pl.when(j + 1 < half)
                    def _():
                        gfire(j + 1, (b + 1) % nbuf)

                    pltpu.make_async_copy(
                        table_hbm.at[src_v.at[j]], rows_v.at[b], sem).wait()
                    pltpu.sync_copy(rows_v.at[b], agg_sh.at[dst_v.at[j]],
                                    add=True)
                return carry

            lax.fori_loop(0, half // nbuf, outer, 0)
        plsc.subcore_barrier()
        pltpu.sync_copy(agg_sh.at[pl.ds(s * _RZ, _RZ)],
                        out_hbm.at[c, pl.ds(s * _RZ, _RZ)])

    return k(table, srcp, dstp, zeros)


def _tc_degree(dstp):
    bg = 16
    steps = _NT * _RPT // bg

    def body(d_ref, out_ref):
        i = pl.program_id(0)

        @pl.when(i == 0)
        def _():
            out_ref[...] = jnp.zeros_like(out_ref)

        subl = lax.broadcasted_iota(jnp.int32, (_D, _D), 0)
        acc = jnp.zeros((_D, _D), jnp.float32)
        for g in range(bg):
            d = d_ref[g:g + 1, :]
            q = d // _D
            r = d - q * _D
            ohqT = (q == subl).astype(jnp.float32)
            ohrT = (r == subl).astype(jnp.float32)
            acc += lax.dot_general(ohqT, ohrT, (((1,), (1,)), ((), ())),
                                   preferred_element_type=jnp.float32)
        out_ref[...] += acc

        @pl.when(i == steps - 1)
        def _():
            out_ref[...] = jnp.maximum(out_ref[...], 1.0)

    return pl.pallas_call(
        body,
        grid=(steps,),
        in_specs=[pl.BlockSpec((bg, _CH), lambda i: (i, 0))],
        out_specs=pl.BlockSpec((_D, _D), lambda i: (0, 0)),
        out_shape=jax.ShapeDtypeStruct((_D, _D), jnp.float32),
    )(dstp)


def _tc_first(x, WsT, b, WnT):
    bm = 2000
    grid = _N // bm

    def body(x_ref, ws_ref, b_ref, wn_ref, self_ref, tab_ref):
        h = x_ref[...]
        self_ref[...] = (
            jnp.dot(h, ws_ref[...], preferred_element_type=jnp.float32)
            + b_ref[...]
        )
        tab_ref[...] = jnp.dot(h, wn_ref[...], preferred_element_type=jnp.float32)

    return pl.pallas_call(
        body,
        grid=(grid,),
        in_specs=[
            pl.BlockSpec((bm, _D), lambda i: (i, 0)),
            pl.BlockSpec((_D, _D), lambda i: (0, 0)),
            pl.BlockSpec((1, _D), lambda i: (0, 0)),
            pl.BlockSpec((_D, _D), lambda i: (0, 0)),
        ],
        out_specs=[
            pl.BlockSpec((bm, _D), lambda i: (i, 0)),
            pl.BlockSpec((bm, _D), lambda i: (i, 0)),
        ],
        out_shape=[
            jax.ShapeDtypeStruct((_N, _D), jnp.float32),
            jax.ShapeDtypeStruct((_N, _D), jnp.float32),
        ],
    )(x, WsT, b, WnT)


def _tc_mid(self1, a0, a1, deg, WsT, b, WnT):
    bm = 2000
    grid = _N // bm

    def body(s_ref, a0_ref, a1_ref, deg_ref, ws_ref, b_ref, wn_ref,
             self_ref, tab_ref):
        agg = a0_ref[...] + a1_ref[...]
        h1 = jnp.maximum(s_ref[...] + agg / deg_ref[...], 0.0)
        self_ref[...] = (
            jnp.dot(h1, ws_ref[...], preferred_element_type=jnp.float32)
            + b_ref[...]
        )
        tab_ref[...] = jnp.dot(h1, wn_ref[...], preferred_element_type=jnp.float32)

    return pl.pallas_call(
        body,
        grid=(grid,),
        in_specs=[
            pl.BlockSpec((bm, _D), lambda i: (i, 0)),
            pl.BlockSpec((bm, _D), lambda i: (i, 0)),
            pl.BlockSpec((bm, _D), lambda i: (i, 0)),
            pl.BlockSpec((bm, 1), lambda i: (i, 0)),
            pl.BlockSpec((_D, _D), lambda i: (0, 0)),
            pl.BlockSpec((1, _D), lambda i: (0, 0)),
            pl.BlockSpec((_D, _D), lambda i: (0, 0)),
        ],
        out_specs=[
            pl.BlockSpec((bm, _D), lambda i: (i, 0)),
            pl.BlockSpec((bm, _D), lambda i: (i, 0)),
        ],
        out_shape=[
            jax.ShapeDtypeStruct((_N, _D), jnp.float32),
            jax.ShapeDtypeStruct((_N, _D), jnp.float32),
        ],
    )(self1, a0, a1, deg, WsT, b, WnT)


def _tc_final(self2, a0, a1, deg):
    bm = 2000
    grid = _N // bm

    def body(s_ref, a0_ref, a1_ref, deg_ref, out_ref):
        agg = a0_ref[...] + a1_ref[...]
        out_ref[...] = s_ref[...] + agg / deg_ref[...]

    return pl.pallas_call(
        body,
        grid=(grid,),
        in_specs=[
            pl.BlockSpec((bm, _D), lambda i: (i, 0)),
            pl.BlockSpec((bm, _D), lambda i: (i, 0)),
            pl.BlockSpec((bm, _D), lambda i: (i, 0)),
            pl.BlockSpec((bm, 1), lambda i: (i, 0)),
        ],
        out_specs=pl.BlockSpec((bm, _D), lambda i: (i, 0)),
        out_shape=jax.ShapeDtypeStruct((_N, _D), jnp.float32),
    )(self2, a0, a1, deg)


def kernel(x, edge_index, W_self1, b_self1, W_neigh1, W_self2, b_self2, W_neigh2):
    src = edge_index[0]
    dst = edge_index[1]
    pad = _EPAD - _E
    srcp = jnp.concatenate([src, jnp.zeros((pad,), jnp.int32)]).reshape(
        _NT * _RPT, _CH)
    dstp = jnp.concatenate([dst, jnp.full((pad,), _N, jnp.int32)]).reshape(
        _NT * _RPT, _CH)
    zeros = jnp.zeros((_NPAD, _D), jnp.float32)

    deg2d = _tc_degree(dstp)
    deg = deg2d.reshape(_D * _D)[:_N].reshape(_N, 1)

    self1, tab1 = _tc_first(x, W_self1.T, b_self1.reshape(1, _D), W_neigh1.T)
    aggs1 = _sc_scatter(tab1, srcp, dstp, zeros)
    self2, tab2 = _tc_mid(self1, aggs1[0, :_N], aggs1[1, :_N], deg,
                          W_self2.T, b_self2.reshape(1, _D), W_neigh2.T)
    aggs2 = _sc_scatter(tab2, srcp, dstp, zeros)
    return _tc_final(self2, aggs2[0, :_N], aggs2[1, :_N], deg)
